# Initial kernel scaffold; baseline (speedup 1.0000x reference)
import jax
import jax.numpy as jnp
from jax.experimental import pallas as pl


def kernel(X, pair_v, pair_e, W0, b0, g0, be0, W1, b1, g1, be1, W2, b2, g2, be2, Wa1, ba1, Wa2, ba2, Wf1, bf1, Wf2, bf2, Wo, bo, gf, bef):
    raise NotImplementedError("write your pallas kernel here")


# speedup vs baseline: 5.9599x; 5.9599x over previous
import functools

import jax
import jax.numpy as jnp
from jax import lax
from jax.experimental import pallas as pl
from jax.experimental.pallas import tpu as pltpu
from jax.experimental.pallas import tpu_sc as plsc

N = 10000
E = 5000
P = 320000
H = 128

NC = 2
NS = 16
NW = NC * NS

PPT = P // NW
CK = 80
CH = PPT // CK

E_PAD = 5120
N_PAD = 10240
RPT_E = E_PAD // NS
RPT_V = N_PAD // NS

_mesh = plsc.VectorSubcoreMesh(core_axis_name="c", subcore_axis_name="s")


def _wid():
    return lax.axis_index("s") * NC + lax.axis_index("c")


def _make_count(s_pad, rpt):
    @functools.partial(
        pl.kernel,
        out_type=jax.ShapeDtypeStruct((NC * s_pad, H), jnp.float32),
        mesh=_mesh,
        scratch_types=[
            pltpu.VMEM((CH, CK), jnp.int32),
            pltpu.VMEM((CK, H), jnp.float32),
            pltpu.VMEM_SHARED((s_pad, H), jnp.float32),
        ],
    )
    def count(i3, ones_hbm, zer_hbm, out, sidx, ones_v, acc):
        cid = lax.axis_index("c")
        sid = lax.axis_index("s")
        pltpu.sync_copy(i3.at[_wid()], sidx)
        pltpu.sync_copy(ones_hbm, ones_v)
        pltpu.sync_copy(zer_hbm.at[pl.ds(0, rpt)], acc.at[pl.ds(sid * rpt, rpt)])
        plsc.subcore_barrier()

        def body(j, _):
            pltpu.sync_copy(ones_v, acc.at[sidx.at[j]], add=True)
            return _

        lax.fori_loop(0, CH, body, None)
        plsc.subcore_barrier()
        pltpu.sync_copy(acc.at[pl.ds(sid * rpt, rpt)],
                        out.at[pl.ds(cid * s_pad + sid * rpt, rpt)])

    return count


_count_e = _make_count(E_PAD, RPT_E)
_count_v = _make_count(N_PAD, RPT_V)


def _make_hop(s_pad, rpt):
    @functools.partial(
        pl.kernel,
        out_type=jax.ShapeDtypeStruct((NC * s_pad, H), jnp.float32),
        mesh=_mesh,
        scratch_types=[
            pltpu.VMEM((CH, CK), jnp.int32),
            pltpu.VMEM((CH, CK), jnp.int32),
            pltpu.VMEM((CK, H), jnp.float32),
            pltpu.VMEM_SHARED((s_pad, H), jnp.float32),
            pltpu.SemaphoreType.DMA,
        ],
    )
    def hop(g3, s3, table, zer_hbm, out, gidx, sidx, rows, acc, sem):
        cid = lax.axis_index("c")
        sid = lax.axis_index("s")
        wid = _wid()
        pltpu.sync_copy(g3.at[wid], gidx)
        pltpu.sync_copy(s3.at[wid], sidx)
        pltpu.sync_copy(zer_hbm.at[pl.ds(0, rpt)], acc.at[pl.ds(sid * rpt, rpt)])
        plsc.subcore_barrier()

        def body(j, _):
            pltpu.async_copy(table.at[gidx.at[j]], rows, sem).wait()
            pltpu.sync_copy(rows, acc.at[sidx.at[j]], add=True)
            return _

        lax.fori_loop(0, CH, body, None)
        plsc.subcore_barrier()
        pltpu.sync_copy(acc.at[pl.ds(sid * rpt, rpt)],
                        out.at[pl.ds(cid * s_pad + sid * rpt, rpt)])

    return hop


_hop_v2e = _make_hop(E_PAD, RPT_E)
_hop_e2v = _make_hop(N_PAD, RPT_V)


def _bn_body(z, g, be):
    mu = jnp.mean(z, axis=0, keepdims=True)
    d = z - mu
    var = jnp.mean(d * d, axis=0, keepdims=True)
    return d * lax.rsqrt(var + 1e-5) * g + be


def _tc_dense0(x_ref, w_ref, b_ref, g_ref, be_ref, o_ref):
    z = jnp.dot(x_ref[...], w_ref[...], preferred_element_type=jnp.float32)
    o_ref[...] = _bn_body(z + b_ref[...], g_ref[...], be_ref[...])


def _tc_dense(vp_ref, vc_ref, w_ref, b_ref, g_ref, be_ref, o_ref):
    v0 = vp_ref[pl.ds(0, N), :]
    v1 = vp_ref[pl.ds(N_PAD, N), :]
    c0 = vc_ref[pl.ds(0, N), :]
    c1 = vc_ref[pl.ds(N_PAD, N), :]
    rv = 1.0 / jnp.maximum((c0 + c1)[:, :1], 1.0)
    x = jnp.maximum((v0 + v1) * rv, 0.0)
    z = jnp.dot(x, w_ref[...], preferred_element_type=jnp.float32)
    o_ref[...] = _bn_body(z + b_ref[...], g_ref[...], be_ref[...])


def _tc_ecombine(ep_ref, ec_ref, o_ref):
    e0 = ep_ref[pl.ds(0, E_PAD), :]
    e1 = ep_ref[pl.ds(E_PAD, E_PAD), :]
    c0 = ec_ref[pl.ds(0, E_PAD), :]
    c1 = ec_ref[pl.ds(E_PAD, E_PAD), :]
    re = 1.0 / jnp.maximum((c0 + c1)[:, :1], 1.0)
    o_ref[...] = (e0 + e1) * re


def _tc_head(vp_ref, vc_ref, wa1_ref, ba1_ref, wa2_ref, ba2_ref,
             wf1_ref, bf1_ref, wf2_ref, bf2_ref, wo_ref, bo_ref,
             gf_ref, bef_ref, score_ref, att_ref):
    v0 = vp_ref[pl.ds(0, N), :]
    v1 = vp_ref[pl.ds(N_PAD, N), :]
    c0 = vc_ref[pl.ds(0, N), :]
    c1 = vc_ref[pl.ds(N_PAD, N), :]
    rv = 1.0 / jnp.maximum((c0 + c1)[:, :1], 1.0)
    x = jnp.maximum((v0 + v1) * rv, 0.0)
    t = jnp.tanh(jnp.dot(x, wa1_ref[...], preferred_element_type=jnp.float32)
                 + ba1_ref[...])
    att = jax.nn.sigmoid(jnp.dot(t, wa2_ref[...], preferred_element_type=jnp.float32)
                         + ba2_ref[...])
    xw = jnp.maximum(x * att, 0.0)
    xw = _bn_body(xw, gf_ref[...], bef_ref[...])
    h = jnp.maximum(jnp.dot(xw, wf1_ref[...], preferred_element_type=jnp.float32)
                    + bf1_ref[...], 0.0)
    h = jnp.maximum(jnp.dot(h, wf2_ref[...], preferred_element_type=jnp.float32)
                    + bf2_ref[...], 0.0)
    score_ref[...] = jax.nn.sigmoid(
        jnp.dot(h, wo_ref[...], preferred_element_type=jnp.float32) + bo_ref[...])
    att_ref[...] = att


def _call_tc(body, out_shapes, *args):
    return pl.pallas_call(
        body,
        out_shape=out_shapes,
    )(*args)


def kernel(X, pair_v, pair_e, W0, b0, g0, be0, W1, b1, g1, be1, W2, b2, g2, be2,
           Wa1, ba1, Wa2, ba2, Wf1, bf1, Wf2, bf2, Wo, bo, gf, bef):
    f32 = jnp.float32
    pv3 = pair_v.reshape(NW, CH, CK)
    pe3 = pair_e.reshape(NW, CH, CK)
    onesH = jnp.ones((CK, H), f32)
    zerH = jnp.zeros((RPT_V, H), f32)
    r = lambda a: a.reshape(1, -1)

    ecnt = _count_e(pe3, onesH, zerH)
    vcnt = _count_v(pv3, onesH, zerH)

    y = _call_tc(_tc_dense0, jax.ShapeDtypeStruct((N, H), f32),
                 X, W0, r(b0), r(g0), r(be0))
    for (W, b, g, be) in ((W1, b1, g1, be1), (W2, b2, g2, be2), (None,) * 4):
        ep = _hop_v2e(pv3, pe3, y, zerH)
        xe = _call_tc(_tc_ecombine, jax.ShapeDtypeStruct((E_PAD, H), f32),
                      ep, ecnt)
        vp = _hop_e2v(pe3, pv3, xe, zerH)
        if W is None:
            score, att = _call_tc(
                _tc_head,
                (jax.ShapeDtypeStruct((N, 1), f32),
                 jax.ShapeDtypeStruct((N, 1), f32)),
                vp, vcnt, Wa1, r(ba1), Wa2, r(ba2),
                Wf1, r(bf1), Wf2, r(bf2), Wo, r(bo), r(gf), r(bef))
            return (score, att)
        y = _call_tc(_tc_dense, jax.ShapeDtypeStruct((N, H), f32),
                     vp, vcnt, W, b.reshape(1, -1), g.reshape(1, -1),
                     be.reshape(1, -1))


# speedup vs baseline: 9.5291x; 1.5989x over previous
import functools

import jax
import jax.numpy as jnp
from jax import lax
from jax.experimental import pallas as pl
from jax.experimental.pallas import tpu as pltpu
from jax.experimental.pallas import tpu_sc as plsc

N = 10000
E = 5000
P = 320000
H = 128

NC = 2
NS = 16
NW = NC * NS

PPT = P // NW
CK = 100
CH = PPT // CK

E_PAD = 5120
N_PAD = 10240
RPT_E = E_PAD // NS
RPT_V = N_PAD // NS

_mesh = plsc.VectorSubcoreMesh(core_axis_name="c", subcore_axis_name="s")


def _wid():
    return lax.axis_index("s") * NC + lax.axis_index("c")


def _make_count(s_pad, rpt):
    @functools.partial(
        pl.kernel,
        out_type=jax.ShapeDtypeStruct((NC * s_pad, H), jnp.float32),
        mesh=_mesh,
        scratch_types=[
            pltpu.VMEM((CH, CK), jnp.int32),
            pltpu.VMEM((CK, H), jnp.float32),
            pltpu.VMEM_SHARED((s_pad, H), jnp.float32),
        ],
    )
    def count(i3, ones_hbm, zer_hbm, out, sidx, ones_v, acc):
        cid = lax.axis_index("c")
        sid = lax.axis_index("s")
        pltpu.sync_copy(i3.at[_wid()], sidx)
        pltpu.sync_copy(ones_hbm, ones_v)
        pltpu.sync_copy(zer_hbm.at[pl.ds(0, rpt)], acc.at[pl.ds(sid * rpt, rpt)])
        plsc.subcore_barrier()

        def body(j, _):
            pltpu.sync_copy(ones_v, acc.at[sidx.at[j]], add=True)
            return _

        lax.fori_loop(0, CH, body, None)
        plsc.subcore_barrier()
        pltpu.sync_copy(acc.at[pl.ds(sid * rpt, rpt)],
                        out.at[pl.ds(cid * s_pad + sid * rpt, rpt)])

    return count


_count_e = _make_count(E_PAD, RPT_E)
_count_v = _make_count(N_PAD, RPT_V)


def _make_hop(s_pad, rpt, nsw):
    sch = CH // nsw

    @functools.partial(
        pl.kernel,
        out_type=jax.ShapeDtypeStruct((NC * s_pad, H), jnp.float32),
        mesh=_mesh,
        scratch_types=[
            pltpu.VMEM((sch, CK), jnp.int32),
            pltpu.VMEM((sch, CK), jnp.int32),
            pltpu.VMEM((CK, H), jnp.float32),
            pltpu.VMEM((CK, H), jnp.float32),
            pltpu.VMEM_SHARED((s_pad, H), jnp.float32),
            pltpu.SemaphoreType.DMA,
            pltpu.SemaphoreType.DMA,
        ],
    )
    def hop(g3, s3, table, zer_hbm, out,
            gidx, sidx, rows0, rows1, acc, sem0, sem1):
        cid = lax.axis_index("c")
        sid = lax.axis_index("s")
        wid = _wid()
        pltpu.sync_copy(zer_hbm.at[pl.ds(0, rpt)], acc.at[pl.ds(sid * rpt, rpt)])
        plsc.subcore_barrier()

        for s in range(nsw):
            pltpu.sync_copy(g3.at[wid * nsw + s], gidx)
            pltpu.sync_copy(s3.at[wid * nsw + s], sidx)
            pltpu.async_copy(table.at[gidx.at[0]], rows0, sem0)
            pltpu.async_copy(table.at[gidx.at[1]], rows1, sem1)

            def body(i, _):
                j0 = 2 * i
                pltpu.make_async_copy(table.at[gidx.at[j0]], rows0, sem0).wait()
                pltpu.sync_copy(rows0, acc.at[sidx.at[j0]], add=True)
                pltpu.async_copy(table.at[gidx.at[j0 + 2]], rows0, sem0)
                pltpu.make_async_copy(table.at[gidx.at[j0 + 1]], rows1, sem1).wait()
                pltpu.sync_copy(rows1, acc.at[sidx.at[j0 + 1]], add=True)
                pltpu.async_copy(table.at[gidx.at[j0 + 3]], rows1, sem1)
                return _

            lax.fori_loop(0, sch // 2 - 1, body, None)
            pltpu.make_async_copy(table.at[gidx.at[sch - 2]], rows0, sem0).wait()
            pltpu.sync_copy(rows0, acc.at[sidx.at[sch - 2]], add=True)
            pltpu.make_async_copy(table.at[gidx.at[sch - 1]], rows1, sem1).wait()
            pltpu.sync_copy(rows1, acc.at[sidx.at[sch - 1]], add=True)

        plsc.subcore_barrier()
        pltpu.sync_copy(acc.at[pl.ds(sid * rpt, rpt)],
                        out.at[pl.ds(cid * s_pad + sid * rpt, rpt)])

    return hop


_hop_v2e = _make_hop(E_PAD, RPT_E, 1)
_hop_e2v = _make_hop(N_PAD, RPT_V, 2)


def _bn_body(z, g, be):
    mu = jnp.mean(z, axis=0, keepdims=True)
    d = z - mu
    var = jnp.mean(d * d, axis=0, keepdims=True)
    return d * lax.rsqrt(var + 1e-5) * g + be


def _tc_dense0(x_ref, w_ref, b_ref, g_ref, be_ref, o_ref):
    z = jnp.dot(x_ref[...], w_ref[...], preferred_element_type=jnp.float32)
    o_ref[...] = _bn_body(z + b_ref[...], g_ref[...], be_ref[...])


def _vcombine(vp_ref, vc_ref):
    v0 = vp_ref[pl.ds(0, N), :]
    v1 = vp_ref[pl.ds(N_PAD, N), :]
    c0 = vc_ref[pl.ds(0, N), :]
    c1 = vc_ref[pl.ds(N_PAD, N), :]
    rv = 1.0 / jnp.maximum((c0 + c1)[:, :1], 1.0)
    return jnp.maximum((v0 + v1) * rv, 0.0)


def _tc_dense(vp_ref, vc_ref, w_ref, b_ref, g_ref, be_ref, o_ref):
    x = _vcombine(vp_ref, vc_ref)
    z = jnp.dot(x, w_ref[...], preferred_element_type=jnp.float32)
    o_ref[...] = _bn_body(z + b_ref[...], g_ref[...], be_ref[...])


def _tc_ecombine(ep_ref, ec_ref, o_ref):
    e0 = ep_ref[pl.ds(0, E_PAD), :]
    e1 = ep_ref[pl.ds(E_PAD, E_PAD), :]
    c0 = ec_ref[pl.ds(0, E_PAD), :]
    c1 = ec_ref[pl.ds(E_PAD, E_PAD), :]
    re = 1.0 / jnp.maximum((c0 + c1)[:, :1], 1.0)
    o_ref[...] = (e0 + e1) * re


def _tc_head(vp_ref, vc_ref, wa1_ref, ba1_ref, wa2_ref, ba2_ref,
             wf1_ref, bf1_ref, wf2_ref, bf2_ref, wo_ref, bo_ref,
             gf_ref, bef_ref, score_ref, att_ref):
    x = _vcombine(vp_ref, vc_ref)
    t = jnp.tanh(jnp.dot(x, wa1_ref[...], preferred_element_type=jnp.float32)
                 + ba1_ref[...])
    att = jax.nn.sigmoid(jnp.dot(t, wa2_ref[...], preferred_element_type=jnp.float32)
                         + ba2_ref[...])
    xw = jnp.maximum(x * att, 0.0)
    xw = _bn_body(xw, gf_ref[...], bef_ref[...])
    h = jnp.maximum(jnp.dot(xw, wf1_ref[...], preferred_element_type=jnp.float32)
                    + bf1_ref[...], 0.0)
    h = jnp.maximum(jnp.dot(h, wf2_ref[...], preferred_element_type=jnp.float32)
                    + bf2_ref[...], 0.0)
    score_ref[...] = jax.nn.sigmoid(
        jnp.dot(h, wo_ref[...], preferred_element_type=jnp.float32) + bo_ref[...])
    att_ref[...] = att


def _call_tc(body, out_shapes, *args):
    return pl.pallas_call(body, out_shape=out_shapes)(*args)


def kernel(X, pair_v, pair_e, W0, b0, g0, be0, W1, b1, g1, be1, W2, b2, g2, be2,
           Wa1, ba1, Wa2, ba2, Wf1, bf1, Wf2, bf2, Wo, bo, gf, bef):
    f32 = jnp.float32
    pv3 = pair_v.reshape(NW, CH, CK)
    pe3 = pair_e.reshape(NW, CH, CK)
    pv3s = pair_v.reshape(NW * 2, CH // 2, CK)
    pe3s = pair_e.reshape(NW * 2, CH // 2, CK)
    onesH = jnp.ones((CK, H), f32)
    zerH = jnp.zeros((RPT_V, H), f32)
    r = lambda a: a.reshape(1, -1)

    ecnt = _count_e(pe3, onesH, zerH)
    vcnt = _count_v(pv3, onesH, zerH)

    y = _call_tc(_tc_dense0, jax.ShapeDtypeStruct((N, H), f32),
                 X, W0, r(b0), r(g0), r(be0))
    for (W, b, g, be) in ((W1, b1, g1, be1), (W2, b2, g2, be2), (None,) * 4):
        ep = _hop_v2e(pv3, pe3, y, zerH)
        xe = _call_tc(_tc_ecombine, jax.ShapeDtypeStruct((E_PAD, H), f32),
                      ep, ecnt)
        vp = _hop_e2v(pe3s, pv3s, xe, zerH)
        if W is None:
            score, att = _call_tc(
                _tc_head,
                (jax.ShapeDtypeStruct((N, 1), f32),
                 jax.ShapeDtypeStruct((N, 1), f32)),
                vp, vcnt, Wa1, r(ba1), Wa2, r(ba2),
                Wf1, r(bf1), Wf2, r(bf2), Wo, r(bo), r(gf), r(bef))
            return (score, att)
        y = _call_tc(_tc_dense, jax.ShapeDtypeStruct((N, H), f32),
                     vp, vcnt, W, b.reshape(1, -1), g.reshape(1, -1),
                     be.reshape(1, -1))


# speedup vs baseline: 11.1507x; 1.1702x over previous
import functools

import jax
import jax.numpy as jnp
from jax import lax
from jax.experimental import pallas as pl
from jax.experimental.pallas import tpu as pltpu
from jax.experimental.pallas import tpu_sc as plsc

N = 10000
E = 5000
P = 320000
H = 128

NC = 2
NS = 16
NW = NC * NS

PPT = P // NW
CK = 100
CH = PPT // CK

E_PAD = 5120
N_PAD = 10240
RPT_E = E_PAD // NS
RPT_V = N_PAD // NS

_mesh = plsc.VectorSubcoreMesh(core_axis_name="c", subcore_axis_name="s")


def _wid():
    return lax.axis_index("s") * NC + lax.axis_index("c")


TOT = E_PAD + N_PAD
HPT = 2 * P // NW
CNT_SZ = TOT + H

@functools.partial(
    pl.kernel,
    out_type=jax.ShapeDtypeStruct((NW, TOT), jnp.float32),
    mesh=_mesh,
    scratch_types=[
        pltpu.VMEM((HPT,), jnp.int32),
        pltpu.VMEM((CNT_SZ,), jnp.float32),
    ],
    compiler_params=pltpu.CompilerParams(needs_layout_passes=False),
)
def _sc_counts(idx2, zer_hbm, out, idxb, cnt):
    wid = _wid()
    pltpu.sync_copy(idx2.at[wid], idxb)
    pltpu.sync_copy(zer_hbm, cnt)
    ones16 = jnp.ones((16,), jnp.float32)

    def body(k, _):
        for c in range(8):
            start = pl.multiple_of((8 * k + c) * 16, 16)
            v = idxb[pl.ds(start, 16)]
            plsc.addupdate_scatter(cnt, [v], ones16)
        return _

    lax.fori_loop(0, HPT // 128, body, None)
    for c in range((HPT % 128) // 16):
        v = idxb[pl.ds((HPT // 128) * 128 + c * 16, 16)]
        plsc.addupdate_scatter(cnt, [v], ones16)
    pltpu.sync_copy(cnt.at[pl.ds(0, TOT)], out.at[wid])


def _make_hop(s_pad, rpt, nsw):
    sch = CH // nsw

    @functools.partial(
        pl.kernel,
        out_type=jax.ShapeDtypeStruct((NC * s_pad, H), jnp.float32),
        mesh=_mesh,
        scratch_types=[
            pltpu.VMEM((sch, CK), jnp.int32),
            pltpu.VMEM((sch, CK), jnp.int32),
            pltpu.VMEM((CK, H), jnp.float32),
            pltpu.VMEM((CK, H), jnp.float32),
            pltpu.VMEM_SHARED((s_pad, H), jnp.float32),
            pltpu.SemaphoreType.DMA,
            pltpu.SemaphoreType.DMA,
        ],
    )
    def hop(g3, s3, table, zer_hbm, out,
            gidx, sidx, rows0, rows1, acc, sem0, sem1):
        cid = lax.axis_index("c")
        sid = lax.axis_index("s")
        wid = _wid()
        pltpu.sync_copy(zer_hbm.at[pl.ds(0, rpt)], acc.at[pl.ds(sid * rpt, rpt)])
        plsc.subcore_barrier()

        for s in range(nsw):
            pltpu.sync_copy(g3.at[wid * nsw + s], gidx)
            pltpu.sync_copy(s3.at[wid * nsw + s], sidx)
            pltpu.async_copy(table.at[gidx.at[0]], rows0, sem0)
            pltpu.async_copy(table.at[gidx.at[1]], rows1, sem1)

            def body(i, _):
                j0 = 2 * i
                pltpu.make_async_copy(table.at[gidx.at[j0]], rows0, sem0).wait()
                pltpu.sync_copy(rows0, acc.at[sidx.at[j0]], add=True)
                pltpu.async_copy(table.at[gidx.at[j0 + 2]], rows0, sem0)
                pltpu.make_async_copy(table.at[gidx.at[j0 + 1]], rows1, sem1).wait()
                pltpu.sync_copy(rows1, acc.at[sidx.at[j0 + 1]], add=True)
                pltpu.async_copy(table.at[gidx.at[j0 + 3]], rows1, sem1)
                return _

            lax.fori_loop(0, sch // 2 - 1, body, None)
            pltpu.make_async_copy(table.at[gidx.at[sch - 2]], rows0, sem0).wait()
            pltpu.sync_copy(rows0, acc.at[sidx.at[sch - 2]], add=True)
            pltpu.make_async_copy(table.at[gidx.at[sch - 1]], rows1, sem1).wait()
            pltpu.sync_copy(rows1, acc.at[sidx.at[sch - 1]], add=True)

        plsc.subcore_barrier()
        pltpu.sync_copy(acc.at[pl.ds(sid * rpt, rpt)],
                        out.at[pl.ds(cid * s_pad + sid * rpt, rpt)])

    return hop


_hop_v2e = _make_hop(E_PAD, RPT_E, 1)
_hop_e2v = _make_hop(N_PAD, RPT_V, 2)


def _bn_body(z, g, be):
    mu = jnp.mean(z, axis=0, keepdims=True)
    d = z - mu
    var = jnp.mean(d * d, axis=0, keepdims=True)
    return d * lax.rsqrt(var + 1e-5) * g + be


def _tc_dense0(x_ref, w_ref, b_ref, g_ref, be_ref, o_ref):
    z = jnp.dot(x_ref[...], w_ref[...], preferred_element_type=jnp.float32)
    o_ref[...] = _bn_body(z + b_ref[...], g_ref[...], be_ref[...])


def _vcombine(vp_ref, h_ref):
    v0 = vp_ref[pl.ds(0, N), :]
    v1 = vp_ref[pl.ds(N_PAD, N), :]
    c = jnp.sum(h_ref[...], axis=0, keepdims=True)[:, E_PAD:E_PAD + N]
    rv = 1.0 / jnp.maximum(jnp.swapaxes(c, 0, 1), 1.0)
    return jnp.maximum((v0 + v1) * rv, 0.0)


def _tc_dense(vp_ref, vc_ref, w_ref, b_ref, g_ref, be_ref, o_ref):
    x = _vcombine(vp_ref, vc_ref)
    z = jnp.dot(x, w_ref[...], preferred_element_type=jnp.float32)
    o_ref[...] = _bn_body(z + b_ref[...], g_ref[...], be_ref[...])


def _tc_ecombine(ep_ref, h_ref, o_ref):
    e0 = ep_ref[pl.ds(0, E_PAD), :]
    e1 = ep_ref[pl.ds(E_PAD, E_PAD), :]
    c = jnp.sum(h_ref[...], axis=0, keepdims=True)[:, :E_PAD]
    re = 1.0 / jnp.maximum(jnp.swapaxes(c, 0, 1), 1.0)
    o_ref[...] = (e0 + e1) * re


def _tc_head(vp_ref, vc_ref, wa1_ref, ba1_ref, wa2_ref, ba2_ref,
             wf1_ref, bf1_ref, wf2_ref, bf2_ref, wo_ref, bo_ref,
             gf_ref, bef_ref, score_ref, att_ref):
    x = _vcombine(vp_ref, vc_ref)
    t = jnp.tanh(jnp.dot(x, wa1_ref[...], preferred_element_type=jnp.float32)
                 + ba1_ref[...])
    att = jax.nn.sigmoid(jnp.dot(t, wa2_ref[...], preferred_element_type=jnp.float32)
                         + ba2_ref[...])
    xw = jnp.maximum(x * att, 0.0)
    xw = _bn_body(xw, gf_ref[...], bef_ref[...])
    h = jnp.maximum(jnp.dot(xw, wf1_ref[...], preferred_element_type=jnp.float32)
                    + bf1_ref[...], 0.0)
    h = jnp.maximum(jnp.dot(h, wf2_ref[...], preferred_element_type=jnp.float32)
                    + bf2_ref[...], 0.0)
    score_ref[...] = jax.nn.sigmoid(
        jnp.dot(h, wo_ref[...], preferred_element_type=jnp.float32) + bo_ref[...])
    att_ref[...] = att


def _call_tc(body, out_shapes, *args):
    return pl.pallas_call(body, out_shape=out_shapes)(*args)


def kernel(X, pair_v, pair_e, W0, b0, g0, be0, W1, b1, g1, be1, W2, b2, g2, be2,
           Wa1, ba1, Wa2, ba2, Wf1, bf1, Wf2, bf2, Wo, bo, gf, bef):
    f32 = jnp.float32
    pv3 = pair_v.reshape(NW, CH, CK)
    pe3 = pair_e.reshape(NW, CH, CK)
    pv3s = pair_v.reshape(NW * 2, CH // 2, CK)
    pe3s = pair_e.reshape(NW * 2, CH // 2, CK)
    zerH = jnp.zeros((RPT_V, H), f32)
    zerC = jnp.zeros((CNT_SZ,), f32)
    hidx = jnp.concatenate([pair_e, pair_v + E_PAD]).reshape(NW, HPT)
    r = lambda a: a.reshape(1, -1)

    hist = _sc_counts(hidx, zerC)

    y = _call_tc(_tc_dense0, jax.ShapeDtypeStruct((N, H), f32),
                 X, W0, r(b0), r(g0), r(be0))
    for (W, b, g, be) in ((W1, b1, g1, be1), (W2, b2, g2, be2), (None,) * 4):
        ep = _hop_v2e(pv3, pe3, y, zerH)
        xe = _call_tc(_tc_ecombine, jax.ShapeDtypeStruct((E_PAD, H), f32),
                      ep, hist)
        vp = _hop_e2v(pe3s, pv3s, xe, zerH)
        if W is None:
            score, att = _call_tc(
                _tc_head,
                (jax.ShapeDtypeStruct((N, 1), f32),
                 jax.ShapeDtypeStruct((N, 1), f32)),
                vp, hist, Wa1, r(ba1), Wa2, r(ba2),
                Wf1, r(bf1), Wf2, r(bf2), Wo, r(bo), r(gf), r(bef))
            return (score, att)
        y = _call_tc(_tc_dense, jax.ShapeDtypeStruct((N, H), f32),
                     vp, hist, W, b.reshape(1, -1), g.reshape(1, -1),
                     be.reshape(1, -1))


# speedup vs baseline: 11.6785x; 1.0473x over previous
import functools

import jax
import jax.numpy as jnp
from jax import lax
from jax.experimental import pallas as pl
from jax.experimental.pallas import tpu as pltpu
from jax.experimental.pallas import tpu_sc as plsc

N = 10000
E = 5000
P = 320000
H = 128

NC = 2
NS = 16
NW = NC * NS

PPT = P // NW
CK = 125
CH = PPT // CK

E_PAD = 5120
N_PAD = 10112
RPT_E = E_PAD // NS
RPT_V = N_PAD // NS

_mesh = plsc.VectorSubcoreMesh(core_axis_name="c", subcore_axis_name="s")


def _wid():
    return lax.axis_index("s") * NC + lax.axis_index("c")


TOT = E_PAD + N_PAD
HPT = 2 * P // NW
CNT_SZ = TOT + H

@functools.partial(
    pl.kernel,
    out_type=jax.ShapeDtypeStruct((NW, TOT), jnp.float32),
    mesh=_mesh,
    scratch_types=[
        pltpu.VMEM((HPT,), jnp.int32),
        pltpu.VMEM((CNT_SZ,), jnp.float32),
    ],
    compiler_params=pltpu.CompilerParams(needs_layout_passes=False),
)
def _sc_counts(idx2, zer_hbm, out, idxb, cnt):
    wid = _wid()
    pltpu.sync_copy(idx2.at[wid], idxb)
    pltpu.sync_copy(zer_hbm, cnt)
    ones16 = jnp.ones((16,), jnp.float32)

    def body(k, _):
        for c in range(8):
            start = pl.multiple_of((8 * k + c) * 16, 16)
            v = idxb[pl.ds(start, 16)]
            plsc.addupdate_scatter(cnt, [v], ones16)
        return _

    lax.fori_loop(0, HPT // 128, body, None)
    for c in range((HPT % 128) // 16):
        v = idxb[pl.ds((HPT // 128) * 128 + c * 16, 16)]
        plsc.addupdate_scatter(cnt, [v], ones16)
    pltpu.sync_copy(cnt.at[pl.ds(0, TOT)], out.at[wid])


def _make_hop(s_pad, rpt, nsw):
    sch = CH // nsw

    @functools.partial(
        pl.kernel,
        out_type=jax.ShapeDtypeStruct((NC * s_pad, H), jnp.float32),
        mesh=_mesh,
        scratch_types=[
            pltpu.VMEM((sch, CK), jnp.int32),
            pltpu.VMEM((sch, CK), jnp.int32),
            pltpu.VMEM((CK, H), jnp.float32),
            pltpu.VMEM((CK, H), jnp.float32),
            pltpu.VMEM_SHARED((s_pad, H), jnp.float32),
            pltpu.SemaphoreType.DMA,
            pltpu.SemaphoreType.DMA,
            pltpu.SemaphoreType.DMA,
        ],
    )
    def hop(g3, s3, table, zer_hbm, out,
            gidx, sidx, rows0, rows1, acc, sem0, sem1, zsem):
        cid = lax.axis_index("c")
        sid = lax.axis_index("s")
        wid = _wid()
        pltpu.async_copy(zer_hbm.at[pl.ds(0, rpt)],
                         acc.at[pl.ds(sid * rpt, rpt)], zsem)
        pltpu.async_copy(g3.at[wid * nsw], gidx, sem0)
        pltpu.async_copy(s3.at[wid * nsw], sidx, sem1)
        pltpu.make_async_copy(g3.at[wid * nsw], gidx, sem0).wait()
        pltpu.make_async_copy(s3.at[wid * nsw], sidx, sem1).wait()
        pltpu.make_async_copy(zer_hbm.at[pl.ds(0, rpt)],
                              acc.at[pl.ds(sid * rpt, rpt)], zsem).wait()
        plsc.subcore_barrier()

        for s in range(nsw):
            if s > 0:
                pltpu.sync_copy(g3.at[wid * nsw + s], gidx)
                pltpu.sync_copy(s3.at[wid * nsw + s], sidx)
            pltpu.async_copy(table.at[gidx.at[0]], rows0, sem0)
            pltpu.async_copy(table.at[gidx.at[1]], rows1, sem1)

            def body(i, _):
                j0 = 2 * i
                pltpu.make_async_copy(table.at[gidx.at[j0]], rows0, sem0).wait()
                pltpu.sync_copy(rows0, acc.at[sidx.at[j0]], add=True)
                pltpu.async_copy(table.at[gidx.at[j0 + 2]], rows0, sem0)
                pltpu.make_async_copy(table.at[gidx.at[j0 + 1]], rows1, sem1).wait()
                pltpu.sync_copy(rows1, acc.at[sidx.at[j0 + 1]], add=True)
                pltpu.async_copy(table.at[gidx.at[j0 + 3]], rows1, sem1)
                return _

            lax.fori_loop(0, sch // 2 - 1, body, None)
            pltpu.make_async_copy(table.at[gidx.at[sch - 2]], rows0, sem0).wait()
            pltpu.sync_copy(rows0, acc.at[sidx.at[sch - 2]], add=True)
            pltpu.make_async_copy(table.at[gidx.at[sch - 1]], rows1, sem1).wait()
            pltpu.sync_copy(rows1, acc.at[sidx.at[sch - 1]], add=True)

        plsc.subcore_barrier()
        pltpu.sync_copy(acc.at[pl.ds(sid * rpt, rpt)],
                        out.at[pl.ds(cid * s_pad + sid * rpt, rpt)])

    return hop


_hop_v2e = _make_hop(E_PAD, RPT_E, 1)
_hop_e2v = _make_hop(N_PAD, RPT_V, 2)


def _bn_body(z, g, be):
    mu = jnp.mean(z, axis=0, keepdims=True)
    d = z - mu
    var = jnp.mean(d * d, axis=0, keepdims=True)
    return d * lax.rsqrt(var + 1e-5) * g + be


def _tc_dense0(x_ref, w_ref, b_ref, g_ref, be_ref, o_ref):
    z = jnp.dot(x_ref[...], w_ref[...], preferred_element_type=jnp.float32)
    o_ref[...] = _bn_body(z + b_ref[...], g_ref[...], be_ref[...])


def _vcombine(vp_ref, h_ref):
    v0 = vp_ref[pl.ds(0, N), :]
    v1 = vp_ref[pl.ds(N_PAD, N), :]
    c = jnp.sum(h_ref[...], axis=0, keepdims=True)[:, E_PAD:E_PAD + N]
    rv = 1.0 / jnp.maximum(jnp.swapaxes(c, 0, 1), 1.0)
    return jnp.maximum((v0 + v1) * rv, 0.0)


def _tc_dense(vp_ref, vc_ref, w_ref, b_ref, g_ref, be_ref, o_ref):
    x = _vcombine(vp_ref, vc_ref)
    z = jnp.dot(x, w_ref[...], preferred_element_type=jnp.float32)
    o_ref[...] = _bn_body(z + b_ref[...], g_ref[...], be_ref[...])


def _tc_ecombine(ep_ref, h_ref, o_ref):
    e0 = ep_ref[pl.ds(0, E_PAD), :]
    e1 = ep_ref[pl.ds(E_PAD, E_PAD), :]
    c = jnp.sum(h_ref[...], axis=0, keepdims=True)[:, :E_PAD]
    re = 1.0 / jnp.maximum(jnp.swapaxes(c, 0, 1), 1.0)
    o_ref[...] = (e0 + e1) * re


def _tc_head(vp_ref, vc_ref, wa1_ref, ba1_ref, wa2_ref, ba2_ref,
             wf1_ref, bf1_ref, wf2_ref, bf2_ref, wo_ref, bo_ref,
             gf_ref, bef_ref, score_ref, att_ref):
    x = _vcombine(vp_ref, vc_ref)
    t = jnp.tanh(jnp.dot(x, wa1_ref[...], preferred_element_type=jnp.float32)
                 + ba1_ref[...])
    att = jax.nn.sigmoid(jnp.dot(t, wa2_ref[...], preferred_element_type=jnp.float32)
                         + ba2_ref[...])
    xw = jnp.maximum(x * att, 0.0)
    xw = _bn_body(xw, gf_ref[...], bef_ref[...])
    h = jnp.maximum(jnp.dot(xw, wf1_ref[...], preferred_element_type=jnp.float32)
                    + bf1_ref[...], 0.0)
    h = jnp.maximum(jnp.dot(h, wf2_ref[...], preferred_element_type=jnp.float32)
                    + bf2_ref[...], 0.0)
    score_ref[...] = jax.nn.sigmoid(
        jnp.dot(h, wo_ref[...], preferred_element_type=jnp.float32) + bo_ref[...])
    att_ref[...] = att


def _call_tc(body, out_shapes, *args):
    return pl.pallas_call(body, out_shape=out_shapes)(*args)


def kernel(X, pair_v, pair_e, W0, b0, g0, be0, W1, b1, g1, be1, W2, b2, g2, be2,
           Wa1, ba1, Wa2, ba2, Wf1, bf1, Wf2, bf2, Wo, bo, gf, bef):
    f32 = jnp.float32
    pv3 = pair_v.reshape(NW, CH, CK)
    pe3 = pair_e.reshape(NW, CH, CK)
    pv3s = pair_v.reshape(NW * 2, CH // 2, CK)
    pe3s = pair_e.reshape(NW * 2, CH // 2, CK)
    zerH = jnp.zeros((RPT_V, H), f32)
    zerC = jnp.zeros((CNT_SZ,), f32)
    hidx = jnp.concatenate([pair_e, pair_v + E_PAD]).reshape(NW, HPT)
    r = lambda a: a.reshape(1, -1)

    hist = _sc_counts(hidx, zerC)

    y = _call_tc(_tc_dense0, jax.ShapeDtypeStruct((N, H), f32),
                 X, W0, r(b0), r(g0), r(be0))
    for (W, b, g, be) in ((W1, b1, g1, be1), (W2, b2, g2, be2), (None,) * 4):
        ep = _hop_v2e(pv3, pe3, y, zerH)
        xe = _call_tc(_tc_ecombine, jax.ShapeDtypeStruct((E_PAD, H), f32),
                      ep, hist)
        vp = _hop_e2v(pe3s, pv3s, xe, zerH)
        if W is None:
            score, att = _call_tc(
                _tc_head,
                (jax.ShapeDtypeStruct((N, 1), f32),
                 jax.ShapeDtypeStruct((N, 1), f32)),
                vp, hist, Wa1, r(ba1), Wa2, r(ba2),
                Wf1, r(bf1), Wf2, r(bf2), Wo, r(bo), r(gf), r(bef))
            return (score, att)
        y = _call_tc(_tc_dense, jax.ShapeDtypeStruct((N, H), f32),
                     vp, hist, W, b.reshape(1, -1), g.reshape(1, -1),
                     be.reshape(1, -1))


# speedup vs baseline: 11.8250x; 1.0125x over previous
import functools

import jax
import jax.numpy as jnp
from jax import lax
from jax.experimental import pallas as pl
from jax.experimental.pallas import tpu as pltpu
from jax.experimental.pallas import tpu_sc as plsc

N = 10000
E = 5000
P = 320000
H = 128

NC = 2
NS = 16
NW = NC * NS

PPT = P // NW
CK = 125
CH = PPT // CK

E_PAD = 5120
N_PAD = 10112
RPT_E = E_PAD // NS
RPT_V = N_PAD // NS

_mesh = plsc.VectorSubcoreMesh(core_axis_name="c", subcore_axis_name="s")


def _wid():
    return lax.axis_index("s") * NC + lax.axis_index("c")


TOT = E_PAD + N_PAD
HPT = 2 * P // NW
HPTP = ((HPT + 127) // 128) * 128
CNT_SZ = TOT + 2 * H


def _make_hop(s_pad, rpt, nsw, with_hist=False):
    sch = CH // nsw
    out_type = jax.ShapeDtypeStruct((NC * s_pad, H), jnp.float32)
    scratch = [
        pltpu.VMEM((sch, CK), jnp.int32),
        pltpu.VMEM((sch, CK), jnp.int32),
        pltpu.VMEM((CK, H), jnp.float32),
        pltpu.VMEM((CK, H), jnp.float32),
        pltpu.VMEM_SHARED((s_pad, H), jnp.float32),
        pltpu.SemaphoreType.DMA,
        pltpu.SemaphoreType.DMA,
        pltpu.SemaphoreType.DMA,
    ]
    if with_hist:
        out_type = (out_type, jax.ShapeDtypeStruct((NW, TOT), jnp.float32))
        scratch += [pltpu.VMEM((HPTP,), jnp.int32),
                    pltpu.VMEM((CNT_SZ,), jnp.float32)]

    def hop(g3, s3, table, zer_hbm, *rest):
        if with_hist:
            (hidx2, zerc, out, hout,
             gidx, sidx, rows0, rows1, acc, sem0, sem1, zsem, idxb, cnt) = rest
        else:
            (out, gidx, sidx, rows0, rows1, acc, sem0, sem1, zsem) = rest
        cid = lax.axis_index("c")
        sid = lax.axis_index("s")
        wid = _wid()
        pltpu.async_copy(zer_hbm.at[pl.ds(0, rpt)],
                         acc.at[pl.ds(sid * rpt, rpt)], zsem)
        pltpu.async_copy(g3.at[wid * nsw], gidx, sem0)
        pltpu.async_copy(s3.at[wid * nsw], sidx, sem1)
        if with_hist:
            pltpu.sync_copy(hidx2.at[wid], idxb)
            pltpu.sync_copy(zerc, cnt)
        ones16 = jnp.ones((16,), jnp.float32)

        def hist_rows(r0, nr):
            if not with_hist:
                return
            for r in range(nr):
                for c in range(8):
                    start = pl.multiple_of((r0 + r) * 128 + c * 16, 16)
                    v = idxb[pl.ds(start, 16)]
                    plsc.addupdate_scatter(cnt, [v], ones16)

        pltpu.make_async_copy(g3.at[wid * nsw], gidx, sem0).wait()
        pltpu.make_async_copy(s3.at[wid * nsw], sidx, sem1).wait()
        pltpu.make_async_copy(zer_hbm.at[pl.ds(0, rpt)],
                              acc.at[pl.ds(sid * rpt, rpt)], zsem).wait()
        plsc.subcore_barrier()

        for s in range(nsw):
            if s > 0:
                pltpu.sync_copy(g3.at[wid * nsw + s], gidx)
                pltpu.sync_copy(s3.at[wid * nsw + s], sidx)
            pltpu.async_copy(table.at[gidx.at[0]], rows0, sem0)
            pltpu.async_copy(table.at[gidx.at[1]], rows1, sem1)

            def body(i, _):
                j0 = 2 * i
                if with_hist:
                    r0 = pl.multiple_of(4 * i, 4)
                    hist_rows(r0, 4)
                pltpu.make_async_copy(table.at[gidx.at[j0]], rows0, sem0).wait()
                pltpu.sync_copy(rows0, acc.at[sidx.at[j0]], add=True)
                pltpu.async_copy(table.at[gidx.at[j0 + 2]], rows0, sem0)
                pltpu.make_async_copy(table.at[gidx.at[j0 + 1]], rows1, sem1).wait()
                pltpu.sync_copy(rows1, acc.at[sidx.at[j0 + 1]], add=True)
                pltpu.async_copy(table.at[gidx.at[j0 + 3]], rows1, sem1)
                return _

            lax.fori_loop(0, sch // 2 - 1, body, None)
            pltpu.make_async_copy(table.at[gidx.at[sch - 2]], rows0, sem0).wait()
            pltpu.sync_copy(rows0, acc.at[sidx.at[sch - 2]], add=True)
            pltpu.make_async_copy(table.at[gidx.at[sch - 1]], rows1, sem1).wait()
            pltpu.sync_copy(rows1, acc.at[sidx.at[sch - 1]], add=True)

        if with_hist:
            done = 4 * (sch // 2 - 1)
            hist_rows(done, HPTP // 128 - done)
            pltpu.sync_copy(cnt.at[pl.ds(0, TOT)], hout.at[wid])
        plsc.subcore_barrier()
        pltpu.sync_copy(acc.at[pl.ds(sid * rpt, rpt)],
                        out.at[pl.ds(cid * s_pad + sid * rpt, rpt)])

    kwargs = dict(out_type=out_type, mesh=_mesh, scratch_types=scratch)
    if with_hist:
        kwargs["compiler_params"] = pltpu.CompilerParams(
            needs_layout_passes=False)
    return pl.kernel(hop, **kwargs)


_hop_v2e = _make_hop(E_PAD, RPT_E, 1)
_hop_e2v = _make_hop(N_PAD, RPT_V, 2)
_hop_v2e_hist = _make_hop(E_PAD, RPT_E, 1, with_hist=True)


def _bn_body(z, g, be):
    mu = jnp.mean(z, axis=0, keepdims=True)
    d = z - mu
    var = jnp.mean(d * d, axis=0, keepdims=True)
    return d * lax.rsqrt(var + 1e-5) * g + be


def _tc_dense0(x_ref, w_ref, b_ref, g_ref, be_ref, o_ref):
    z = jnp.dot(x_ref[...], w_ref[...], preferred_element_type=jnp.float32)
    o_ref[...] = _bn_body(z + b_ref[...], g_ref[...], be_ref[...])


def _vcombine(vp_ref, h_ref):
    v0 = vp_ref[pl.ds(0, N), :]
    v1 = vp_ref[pl.ds(N_PAD, N), :]
    c = jnp.sum(h_ref[...], axis=0, keepdims=True)[:, E_PAD:E_PAD + N]
    rv = 1.0 / jnp.maximum(jnp.swapaxes(c, 0, 1), 1.0)
    return jnp.maximum((v0 + v1) * rv, 0.0)


def _tc_dense(vp_ref, vc_ref, w_ref, b_ref, g_ref, be_ref, o_ref):
    x = _vcombine(vp_ref, vc_ref)
    z = jnp.dot(x, w_ref[...], preferred_element_type=jnp.float32)
    o_ref[...] = _bn_body(z + b_ref[...], g_ref[...], be_ref[...])


def _tc_ecombine(ep_ref, h_ref, o_ref):
    e0 = ep_ref[pl.ds(0, E_PAD), :]
    e1 = ep_ref[pl.ds(E_PAD, E_PAD), :]
    c = jnp.sum(h_ref[...], axis=0, keepdims=True)[:, :E_PAD]
    re = 1.0 / jnp.maximum(jnp.swapaxes(c, 0, 1), 1.0)
    o_ref[...] = (e0 + e1) * re


def _tc_head(vp_ref, vc_ref, wa1_ref, ba1_ref, wa2_ref, ba2_ref,
             wf1_ref, bf1_ref, wf2_ref, bf2_ref, wo_ref, bo_ref,
             gf_ref, bef_ref, score_ref, att_ref):
    x = _vcombine(vp_ref, vc_ref)
    t = jnp.tanh(jnp.dot(x, wa1_ref[...], preferred_element_type=jnp.float32)
                 + ba1_ref[...])
    att = jax.nn.sigmoid(jnp.dot(t, wa2_ref[...], preferred_element_type=jnp.float32)
                         + ba2_ref[...])
    xw = jnp.maximum(x * att, 0.0)
    xw = _bn_body(xw, gf_ref[...], bef_ref[...])
    h = jnp.maximum(jnp.dot(xw, wf1_ref[...], preferred_element_type=jnp.float32)
                    + bf1_ref[...], 0.0)
    h = jnp.maximum(jnp.dot(h, wf2_ref[...], preferred_element_type=jnp.float32)
                    + bf2_ref[...], 0.0)
    score_ref[...] = jax.nn.sigmoid(
        jnp.dot(h, wo_ref[...], preferred_element_type=jnp.float32) + bo_ref[...])
    att_ref[...] = att


def _call_tc(body, out_shapes, *args):
    return pl.pallas_call(body, out_shape=out_shapes)(*args)


def kernel(X, pair_v, pair_e, W0, b0, g0, be0, W1, b1, g1, be1, W2, b2, g2, be2,
           Wa1, ba1, Wa2, ba2, Wf1, bf1, Wf2, bf2, Wo, bo, gf, bef):
    f32 = jnp.float32
    pv3 = pair_v.reshape(NW, CH, CK)
    pe3 = pair_e.reshape(NW, CH, CK)
    pv3s = pair_v.reshape(NW * 2, CH // 2, CK)
    pe3s = pair_e.reshape(NW * 2, CH // 2, CK)
    zerH = jnp.zeros((RPT_V, H), f32)
    zerC = jnp.zeros((CNT_SZ,), f32)
    hidx = jnp.concatenate([pair_e, pair_v + E_PAD]).reshape(NW, HPT)
    hidx = jnp.concatenate(
        [hidx, jnp.full((NW, HPTP - HPT), TOT, jnp.int32)], axis=1)
    r = lambda a: a.reshape(1, -1)

    hist = None
    y = _call_tc(_tc_dense0, jax.ShapeDtypeStruct((N, H), f32),
                 X, W0, r(b0), r(g0), r(be0))
    for (W, b, g, be) in ((W1, b1, g1, be1), (W2, b2, g2, be2), (None,) * 4):
        if hist is None:
            ep, hist = _hop_v2e_hist(pv3, pe3, y, zerH, hidx, zerC)
        else:
            ep = _hop_v2e(pv3, pe3, y, zerH)
        xe = _call_tc(_tc_ecombine, jax.ShapeDtypeStruct((E_PAD, H), f32),
                      ep, hist)
        vp = _hop_e2v(pe3s, pv3s, xe, zerH)
        if W is None:
            score, att = _call_tc(
                _tc_head,
                (jax.ShapeDtypeStruct((N, 1), f32),
                 jax.ShapeDtypeStruct((N, 1), f32)),
                vp, hist, Wa1, r(ba1), Wa2, r(ba2),
                Wf1, r(bf1), Wf2, r(bf2), Wo, r(bo), r(gf), r(bef))
            return (score, att)
        y = _call_tc(_tc_dense, jax.ShapeDtypeStruct((N, H), f32),
                     vp, hist, W, b.reshape(1, -1), g.reshape(1, -1),
                     be.reshape(1, -1))
